# baseline (speedup 1.0000x reference)
import functools

import jax
import jax.numpy as jnp
from jax import lax
from jax.experimental import pallas as pl
from jax.experimental.pallas import tpu as pltpu
from jax.experimental.pallas import tpu_sc as plsc

_B = 16384
_L = 50
_D = 64
_LANES = 16
_CHUNK = 128


def _sc_workers():
    try:
        info = plsc.get_sparse_core_info()
        return info.num_cores, info.num_subcores
    except Exception:
        return 2, 16


@functools.partial(jax.jit, static_argnames=("nc", "ns"))
def _trans_rec(user, item_seq_flat, item_seq_len, user_table, item_table, T,
               nc, ns):
    nw = nc * ns
    b_per_w = _B // nw
    n_chunks = b_per_w // _CHUNK
    n_waves = b_per_w // _LANES
    mesh = plsc.VectorSubcoreMesh(
        core_axis_name="c", subcore_axis_name="s", num_cores=nc,
        num_subcores=ns)

    @functools.partial(
        pl.kernel,
        out_type=jax.ShapeDtypeStruct((_B, _D), jnp.float32),
        mesh=mesh,
        compiler_params=pltpu.CompilerParams(use_tc_tiling_on_sc=True),
        scratch_types=[
            pltpu.VMEM((b_per_w,), jnp.int32),
            pltpu.VMEM((b_per_w,), jnp.int32),
            pltpu.VMEM((b_per_w,), jnp.int32),
            pltpu.VMEM((b_per_w,), jnp.int32),
            pltpu.VMEM((2, _LANES, 8, _D), jnp.float32),
            pltpu.VMEM((2, _LANES, 8, _D), jnp.float32),
            pltpu.VMEM((_LANES, _D), jnp.float32),
            pltpu.VMEM((_D,), jnp.float32),
            pltpu.SemaphoreType.DMA,
            pltpu.SemaphoreType.DMA,
        ],
    )
    def body(user_hbm, iseq_hbm, len_hbm, utab_hbm, itab_hbm, t_hbm,
             out_hbm, uidx_v, len_v, fidx_v, last_v, ublk_v, iblk_v,
             outb_v, t_v, sem_a, sem_b):
        wid = lax.axis_index("s") * nc + lax.axis_index("c")
        base = wid * b_per_w

        pltpu.sync_copy(user_hbm.at[pl.ds(base, b_per_w)], uidx_v)
        pltpu.sync_copy(len_hbm.at[pl.ds(base, b_per_w)], len_v)
        pltpu.sync_copy(t_hbm, t_v)

        for j in range(b_per_w // _LANES):
            sl = pl.ds(j * _LANES, _LANES)
            seq_len = len_v[sl]
            row = lax.iota(jnp.int32, _LANES) + (base + j * _LANES)
            fidx_v[sl] = row * _L + seq_len - 1

        copies = []
        for k in range(n_chunks):
            sl = pl.ds(k * _CHUNK, _CHUNK)
            copies.append(pltpu.async_copy(
                iseq_hbm.at[fidx_v.at[sl]], last_v.at[sl], sem_a))
        for c in copies:
            c.wait()

        t_regs = [t_v[pl.ds(d * _LANES, _LANES)]
                  for d in range(_D // _LANES)]

        def fire_wave(w, slot):
            uvec = uidx_v[pl.ds(w * _LANES, _LANES)]
            ivec = last_v[pl.ds(w * _LANES, _LANES)]
            for lane in range(_LANES):
                uoff = pl.multiple_of((uvec[lane] >> 3) * 8, 8)
                ioff = pl.multiple_of((ivec[lane] >> 3) * 8, 8)
                pltpu.async_copy(
                    utab_hbm.at[pl.ds(uoff, 8)], ublk_v.at[slot, lane],
                    sem_b)
                pltpu.async_copy(
                    itab_hbm.at[pl.ds(ioff, 8)], iblk_v.at[slot, lane],
                    sem_b)

        def drain_wave(slot):
            for lane in range(_LANES):
                pltpu.make_async_copy(
                    utab_hbm.at[pl.ds(0, 8)], ublk_v.at[slot, lane],
                    sem_b).wait()
                pltpu.make_async_copy(
                    itab_hbm.at[pl.ds(0, 8)], iblk_v.at[slot, lane],
                    sem_b).wait()

        fire_wave(0, 0)

        def wave_body(g, _):
            slot = lax.rem(g, 2)

            @pl.when(g + 1 < n_waves)
            def _():
                fire_wave(g + 1, 1 - slot)

            drain_wave(slot)

            uvec = uidx_v[pl.ds(g * _LANES, _LANES)]
            ivec = last_v[pl.ds(g * _LANES, _LANES)]
            for lane in range(_LANES):
                ru = uvec[lane] & 7
                ri = ivec[lane] & 7
                for d in range(_D // _LANES):
                    sl = pl.ds(d * _LANES, _LANES)
                    outb_v[lane, sl] = (
                        ublk_v[slot, lane, ru, sl]
                        + iblk_v[slot, lane, ri, sl] + t_regs[d])

            pltpu.sync_copy(
                outb_v, out_hbm.at[pl.ds(base + g * _LANES, _LANES)])
            return 0

        lax.fori_loop(0, n_waves, wave_body, 0)

    return body(user, item_seq_flat, item_seq_len, user_table, item_table, T)


def kernel(user, item_seq, item_seq_len, user_table, item_table, T):
    nc, ns = _sc_workers()
    out = _trans_rec(
        user.astype(jnp.int32),
        item_seq.reshape(-1).astype(jnp.int32),
        item_seq_len.astype(jnp.int32),
        user_table, item_table, T, nc, ns)
    probe = jnp.take(user_table, jnp.zeros((4096,), jnp.int32), axis=0)
    guard = jnp.min(user) < jnp.int32(-2147483647)
    return out.at[0, 0].add(jnp.where(guard, probe.sum(), jnp.float32(0.0)))


# speedup vs baseline: 1.1844x; 1.1844x over previous
import functools

import jax
import jax.numpy as jnp
from jax import lax
from jax.experimental import pallas as pl
from jax.experimental.pallas import tpu as pltpu
from jax.experimental.pallas import tpu_sc as plsc

_B = 16384
_L = 50
_D = 64
_LANES = 16
_CHUNK = 128


def _sc_workers():
    try:
        info = plsc.get_sparse_core_info()
        return info.num_cores, info.num_subcores
    except Exception:
        return 2, 16


@functools.partial(jax.jit, static_argnames=("nc", "ns"))
def _trans_rec(user, item_seq_flat, item_seq_len, user_table, item_table, T,
               nc, ns):
    nw = nc * ns
    b_per_w = _B // nw
    n_chunks = b_per_w // _CHUNK
    n_waves = b_per_w // _LANES
    mesh = plsc.VectorSubcoreMesh(
        core_axis_name="c", subcore_axis_name="s", num_cores=nc,
        num_subcores=ns)

    @functools.partial(
        pl.kernel,
        out_type=jax.ShapeDtypeStruct((_B, _D), jnp.float32),
        mesh=mesh,
        compiler_params=pltpu.CompilerParams(use_tc_tiling_on_sc=True),
        scratch_types=[
            pltpu.VMEM((b_per_w,), jnp.int32),
            pltpu.VMEM((b_per_w,), jnp.int32),
            pltpu.VMEM((b_per_w,), jnp.int32),
            pltpu.VMEM((b_per_w,), jnp.int32),
            pltpu.VMEM((2, _LANES, 8, _D), jnp.float32),
            pltpu.VMEM((2, _LANES, 8, _D), jnp.float32),
            pltpu.VMEM((_LANES, _D), jnp.float32),
            pltpu.VMEM((_D,), jnp.float32),
            pltpu.SemaphoreType.DMA,
            pltpu.SemaphoreType.DMA,
        ],
    )
    def body(user_hbm, iseq_hbm, len_hbm, utab_hbm, itab_hbm, t_hbm,
             out_hbm, uidx_v, len_v, fidx_v, last_v, ublk_v, iblk_v,
             outb_v, t_v, sem_a, sem_b):
        wid = lax.axis_index("s") * nc + lax.axis_index("c")
        base = wid * b_per_w

        pltpu.sync_copy(user_hbm.at[pl.ds(base, b_per_w)], uidx_v)
        pltpu.sync_copy(len_hbm.at[pl.ds(base, b_per_w)], len_v)
        pltpu.sync_copy(t_hbm, t_v)

        for j in range(b_per_w // _LANES):
            sl = pl.ds(j * _LANES, _LANES)
            seq_len = len_v[sl]
            row = lax.iota(jnp.int32, _LANES) + (base + j * _LANES)
            fidx_v[sl] = row * _L + seq_len - 1

        copies = []
        for k in range(n_chunks):
            sl = pl.ds(k * _CHUNK, _CHUNK)
            copies.append(pltpu.async_copy(
                iseq_hbm.at[fidx_v.at[sl]], last_v.at[sl], sem_a))
        for c in copies:
            c.wait()

        t_regs = [t_v[pl.ds(d * _LANES, _LANES)]
                  for d in range(_D // _LANES)]

        def fire_wave(w, slot):
            uvec = uidx_v[pl.ds(w * _LANES, _LANES)]
            ivec = last_v[pl.ds(w * _LANES, _LANES)]
            for lane in range(_LANES):
                uoff = pl.multiple_of((uvec[lane] >> 3) * 8, 8)
                ioff = pl.multiple_of((ivec[lane] >> 3) * 8, 8)
                pltpu.async_copy(
                    utab_hbm.at[pl.ds(uoff, 8)], ublk_v.at[slot, lane],
                    sem_b)
                pltpu.async_copy(
                    itab_hbm.at[pl.ds(ioff, 8)], iblk_v.at[slot, lane],
                    sem_b)

        def drain_wave(slot):
            for lane in range(_LANES):
                pltpu.make_async_copy(
                    utab_hbm.at[pl.ds(0, 8)], ublk_v.at[slot, lane],
                    sem_b).wait()
                pltpu.make_async_copy(
                    itab_hbm.at[pl.ds(0, 8)], iblk_v.at[slot, lane],
                    sem_b).wait()

        fire_wave(0, 0)

        def wave_body(g, _):
            slot = lax.rem(g, 2)

            @pl.when(g + 1 < n_waves)
            def _():
                fire_wave(g + 1, 1 - slot)

            drain_wave(slot)

            uvec = uidx_v[pl.ds(g * _LANES, _LANES)]
            ivec = last_v[pl.ds(g * _LANES, _LANES)]
            for lane in range(_LANES):
                ru = uvec[lane] & 7
                ri = ivec[lane] & 7
                for d in range(_D // _LANES):
                    sl = pl.ds(d * _LANES, _LANES)
                    outb_v[lane, sl] = (
                        ublk_v[slot, lane, ru, sl]
                        + iblk_v[slot, lane, ri, sl] + t_regs[d])

            pltpu.sync_copy(
                outb_v, out_hbm.at[pl.ds(base + g * _LANES, _LANES)])
            return 0

        lax.fori_loop(0, n_waves, wave_body, 0)

    return body(user, item_seq_flat, item_seq_len, user_table, item_table, T)


def kernel(user, item_seq, item_seq_len, user_table, item_table, T):
    nc, ns = _sc_workers()
    return _trans_rec(
        user.astype(jnp.int32),
        item_seq.reshape(-1).astype(jnp.int32),
        item_seq_len.astype(jnp.int32),
        user_table, item_table, T, nc, ns)
